# Initial kernel scaffold; baseline (speedup 1.0000x reference)
import jax
import jax.numpy as jnp
from jax.experimental import pallas as pl


def kernel(x, W1, a1_src, a1_dst, b1, W2, a2_src, a2_dst, b2, edge_index):
    raise NotImplementedError("write your pallas kernel here")


# speedup vs baseline: 17.1129x; 17.1129x over previous
import functools
import jax
import jax.numpy as jnp
from jax import lax
from jax.experimental import pallas as pl
from jax.experimental.pallas import tpu as pltpu
from jax.experimental.pallas import tpu_sc as plsc

NC = 2
NS = 16
NW = NC * NS
L = 16
B = 128
N_PAD = 10240


def _leaky(t):
    return jnp.where(t >= 0, t, 0.2 * t)


def _track_shift(i, grid, es, ed, cv_ref, acc_ref):
    m_es = jnp.max(es)
    m_ed = jnp.max(ed)
    first = i == 0
    acc_ref[0, :] = jnp.where(first, jnp.full((128,), m_es),
                              jnp.maximum(acc_ref[0, :], m_es))
    acc_ref[1, :] = jnp.where(first, jnp.full((128,), m_ed),
                              jnp.maximum(acc_ref[1, :], m_ed))

    @pl.when(i == grid - 1)
    def _():
        cv_ref[...] = _leaky(acc_ref[0, :] + acc_ref[1, :])


def _k1_body(grid, x_ref, w_ref, asrc_ref, adst_ref,
             h_ref, es_ref, ed_ref, cv_ref, acc_ref):
    i = pl.program_id(0)
    h = jnp.dot(x_ref[...], w_ref[...], preferred_element_type=jnp.float32)
    h_ref[...] = h
    es = jnp.sum(h * asrc_ref[...][None, :], axis=1)
    ed = jnp.sum(h * adst_ref[...][None, :], axis=1)
    es_ref[...] = es
    ed_ref[...] = ed
    _track_shift(i, grid, es, ed, cv_ref, acc_ref)


def _dense1(x_pad, W, a_src, a_dst):
    d_in = x_pad.shape[1]
    hid = W.shape[1]
    R = 512
    grid = N_PAD // R
    return pl.pallas_call(
        functools.partial(_k1_body, grid),
        grid=(grid,),
        in_specs=[
            pl.BlockSpec((R, d_in), lambda i: (i, 0)),
            pl.BlockSpec((d_in, hid), lambda i: (0, 0)),
            pl.BlockSpec((hid,), lambda i: (0,)),
            pl.BlockSpec((hid,), lambda i: (0,)),
        ],
        out_specs=[
            pl.BlockSpec((R, hid), lambda i: (i, 0)),
            pl.BlockSpec((R,), lambda i: (i,)),
            pl.BlockSpec((R,), lambda i: (i,)),
            pl.BlockSpec((128,), lambda i: (0,)),
        ],
        out_shape=[
            jax.ShapeDtypeStruct((N_PAD, hid), jnp.float32),
            jax.ShapeDtypeStruct((N_PAD,), jnp.float32),
            jax.ShapeDtypeStruct((N_PAD,), jnp.float32),
            jax.ShapeDtypeStruct((128,), jnp.float32),
        ],
        scratch_shapes=[pltpu.VMEM((2, 128), jnp.float32)],
    )(x_pad, W, a_src, a_dst)


def _sc_agg_body(nb, W, OUTW,
                 h_hbm, es_hbm, ed_hbm, src_hbm, dst_hbm, cv_hbm, out_hbm,
                 srcb, dstb, rows, scaled, esg, edg, eebuf, cbuf,
                 out_acc, es_sh, ed_sh, sem):
    cid = lax.axis_index("c")
    sid = lax.axis_index("s")
    wid = cid * NS + sid
    stripe = N_PAD // NS
    soff = sid * stripe

    pltpu.sync_copy(cv_hbm, cbuf)
    pltpu.sync_copy(es_hbm.at[pl.ds(soff, stripe)],
                    es_sh.at[pl.ds(soff, stripe)])
    pltpu.sync_copy(ed_hbm.at[pl.ds(soff, stripe)],
                    ed_sh.at[pl.ds(soff, stripe)])

    zero = jnp.zeros((L,), jnp.float32)

    def _zrow(j, _):
        for k in range(OUTW // L):
            scaled[j, pl.ds(k * L, L)] = zero
        return 0

    lax.fori_loop(0, B, _zrow, 0)
    for k in range(stripe // B):
        pltpu.sync_copy(scaled, out_acc.at[pl.ds(soff + k * B, B)])
    plsc.subcore_barrier()

    cv = cbuf[pl.ds(0, L)]

    def _block(b, _):
        pltpu.sync_copy(src_hbm.at[wid].at[b], srcb)
        pltpu.sync_copy(dst_hbm.at[wid].at[b], dstb)
        pltpu.async_copy(h_hbm.at[srcb], rows, sem).wait()
        pltpu.sync_copy(es_sh.at[srcb], esg)
        pltpu.sync_copy(ed_sh.at[dstb], edg)
        for g in range(B // L):
            t = esg[pl.ds(g * L, L)] + edg[pl.ds(g * L, L)]
            ee = jnp.exp(_leaky(t) - cv)
            eebuf[pl.ds(g * L, L)] = ee

        def _edge(j, _):
            wv = plsc.load_gather(eebuf, [jnp.full((L,), j, jnp.int32)])
            for k in range(W // L):
                scaled[j, pl.ds(k * L, L)] = rows[j, pl.ds(k * L, L)] * wv
            scaled[j, pl.ds(W, L)] = wv
            return 0

        lax.fori_loop(0, B, _edge, 0)
        pltpu.sync_copy(scaled, out_acc.at[dstb], add=True)
        return 0

    lax.fori_loop(0, nb, _block, 0)
    plsc.subcore_barrier()

    for k in range(stripe // B):
        off = soff + k * B
        pltpu.sync_copy(out_acc.at[pl.ds(off, B)],
                        out_hbm.at[cid].at[pl.ds(off, B)])


def _sc_aggregate(h_pad, es, ed, src3, dst3, cv):
    W = h_pad.shape[1]
    OUTW = W + L
    nb = src3.shape[1]
    mesh = plsc.VectorSubcoreMesh(core_axis_name="c", subcore_axis_name="s")
    body = functools.partial(_sc_agg_body, nb, W, OUTW)
    return pl.kernel(
        body,
        out_type=jax.ShapeDtypeStruct((NC, N_PAD, OUTW), jnp.float32),
        mesh=mesh,
        compiler_params=pltpu.CompilerParams(needs_layout_passes=False,
                                             use_tc_tiling_on_sc=False),
        scratch_types=[
            pltpu.VMEM((B,), jnp.int32),
            pltpu.VMEM((B,), jnp.int32),
            pltpu.VMEM((B, W), jnp.float32),
            pltpu.VMEM((B, OUTW), jnp.float32),
            pltpu.VMEM((B,), jnp.float32),
            pltpu.VMEM((B,), jnp.float32),
            pltpu.VMEM((B,), jnp.float32),
            pltpu.VMEM((128,), jnp.float32),
            pltpu.VMEM_SHARED((N_PAD, OUTW), jnp.float32),
            pltpu.VMEM_SHARED((N_PAD,), jnp.float32),
            pltpu.VMEM_SHARED((N_PAD,), jnp.float32),
            pltpu.SemaphoreType.DMA,
        ],
    )(h_pad, es, ed, src3, dst3, cv)


def _k2_body(grid, s_ref, b1_ref, w2_ref, a2s_ref, a2d_ref,
             h2_ref, es_ref, ed_ref, cv_ref, acc_ref):
    i = pl.program_id(0)
    agg = s_ref[0] + s_ref[1]
    hid = w2_ref.shape[0]
    num = agg[:, :hid]
    den = agg[:, hid:hid + 1]
    den = jnp.where(den > 0, den, 1.0)
    h1 = jax.nn.relu(num / den + b1_ref[...][None, :])
    h2 = jnp.dot(h1, w2_ref[...], preferred_element_type=jnp.float32)
    h2_ref[...] = h2
    es = jnp.sum(h2 * a2s_ref[...][None, :], axis=1)
    ed = jnp.sum(h2 * a2d_ref[...][None, :], axis=1)
    es_ref[...] = es
    ed_ref[...] = ed
    _track_shift(i, grid, es, ed, cv_ref, acc_ref)


def _dense2(S1, b1, W2p, a2s_p, a2d_p):
    hid = W2p.shape[0]
    outw = S1.shape[2]
    lw = W2p.shape[1]
    R = 512
    grid = N_PAD // R
    return pl.pallas_call(
        functools.partial(_k2_body, grid),
        grid=(grid,),
        in_specs=[
            pl.BlockSpec((NC, R, outw), lambda i: (0, i, 0)),
            pl.BlockSpec((hid,), lambda i: (0,)),
            pl.BlockSpec((hid, lw), lambda i: (0, 0)),
            pl.BlockSpec((lw,), lambda i: (0,)),
            pl.BlockSpec((lw,), lambda i: (0,)),
        ],
        out_specs=[
            pl.BlockSpec((R, lw), lambda i: (i, 0)),
            pl.BlockSpec((R,), lambda i: (i,)),
            pl.BlockSpec((R,), lambda i: (i,)),
            pl.BlockSpec((128,), lambda i: (0,)),
        ],
        out_shape=[
            jax.ShapeDtypeStruct((N_PAD, lw), jnp.float32),
            jax.ShapeDtypeStruct((N_PAD,), jnp.float32),
            jax.ShapeDtypeStruct((N_PAD,), jnp.float32),
            jax.ShapeDtypeStruct((128,), jnp.float32),
        ],
        scratch_shapes=[pltpu.VMEM((2, 128), jnp.float32)],
    )(S1, b1, W2p, a2s_p, a2d_p)


def _k3_body(n_label, lw, s_ref, b2_ref, o_ref):
    agg = s_ref[0] + s_ref[1]
    num = agg[:, :n_label]
    den = agg[:, lw:lw + 1]
    den = jnp.where(den > 0, den, 1.0)
    logits = num / den + b2_ref[...][None, :]
    m = jnp.max(logits, axis=1, keepdims=True)
    z = logits - m
    o_ref[...] = z - jnp.log(jnp.sum(jnp.exp(z), axis=1, keepdims=True))


def _dense3(S2, b2, n, n_label, lw):
    outw = S2.shape[2]
    R = 512
    grid = N_PAD // R
    return pl.pallas_call(
        functools.partial(_k3_body, n_label, lw),
        grid=(grid,),
        in_specs=[
            pl.BlockSpec((NC, R, outw), lambda i: (0, i, 0)),
            pl.BlockSpec((n_label,), lambda i: (0,)),
        ],
        out_specs=pl.BlockSpec((R, n_label), lambda i: (i, 0)),
        out_shape=jax.ShapeDtypeStruct((n, n_label), jnp.float32),
    )(S2, b2)


@jax.jit
def kernel(x, W1, a1_src, a1_dst, b1, W2, a2_src, a2_dst, b2, edge_index):
    n, d_in = x.shape
    hid = W1.shape[1]
    n_label = W2.shape[1]
    e = edge_index.shape[1]
    e_tot = e + n

    nb = -(-e_tot // (NW * B))
    e_pad = NW * nb * B
    loop = jnp.arange(n, dtype=jnp.int32)
    src = jnp.concatenate([edge_index[0].astype(jnp.int32), loop])
    dst = jnp.concatenate([edge_index[1].astype(jnp.int32), loop])
    pad_n = jnp.full((e_pad - e_tot,), n, jnp.int32)
    src3 = jnp.concatenate([src, pad_n]).reshape(NW, nb, B)
    dst3 = jnp.concatenate([dst, pad_n]).reshape(NW, nb, B)
    x_pad = jnp.zeros((N_PAD, d_in), jnp.float32).at[:n].set(x)

    h1, es1, ed1, cv1 = _dense1(x_pad, W1, a1_src, a1_dst)
    S1 = _sc_aggregate(h1, es1, ed1, src3, dst3, cv1)

    lw = 48
    W2p = jnp.zeros((hid, lw), jnp.float32).at[:, :n_label].set(W2)
    a2s_p = jnp.zeros((lw,), jnp.float32).at[:n_label].set(a2_src)
    a2d_p = jnp.zeros((lw,), jnp.float32).at[:n_label].set(a2_dst)
    h2, es2, ed2, cv2 = _dense2(S1, b1, W2p, a2s_p, a2d_p)

    S2 = _sc_aggregate(h2, es2, ed2, src3, dst3, cv2)
    return _dense3(S2, b2, n, n_label, lw)


# speedup vs baseline: 17.9298x; 1.0477x over previous
import functools
import jax
import jax.numpy as jnp
from jax import lax
from jax.experimental import pallas as pl
from jax.experimental.pallas import tpu as pltpu
from jax.experimental.pallas import tpu_sc as plsc

NC = 2
NS = 16
NW = NC * NS
L = 16
B = 112
N_PAD = 10240
ZC = 80


def _leaky(t):
    return jnp.where(t >= 0, t, 0.2 * t)


def _track_shift(i, grid, es, ed, cv_ref, acc_ref):
    m_es = jnp.max(es)
    m_ed = jnp.max(ed)
    first = i == 0
    acc_ref[0, :] = jnp.where(first, jnp.full((128,), m_es),
                              jnp.maximum(acc_ref[0, :], m_es))
    acc_ref[1, :] = jnp.where(first, jnp.full((128,), m_ed),
                              jnp.maximum(acc_ref[1, :], m_ed))

    @pl.when(i == grid - 1)
    def _():
        cv_ref[...] = _leaky(acc_ref[0, :] + acc_ref[1, :])


def _k1_body(grid, x_ref, w_ref, asrc_ref, adst_ref,
             h_ref, es_ref, ed_ref, cv_ref, acc_ref):
    i = pl.program_id(0)
    h = jnp.dot(x_ref[...], w_ref[...], preferred_element_type=jnp.float32)
    h_ref[...] = h
    es = jnp.sum(h * asrc_ref[...][None, :], axis=1)
    ed = jnp.sum(h * adst_ref[...][None, :], axis=1)
    es_ref[...] = es
    ed_ref[...] = ed
    _track_shift(i, grid, es, ed, cv_ref, acc_ref)


def _dense1(x_pad, W, a_src, a_dst):
    d_in = x_pad.shape[1]
    hid = W.shape[1]
    R = 512
    grid = N_PAD // R
    return pl.pallas_call(
        functools.partial(_k1_body, grid),
        grid=(grid,),
        in_specs=[
            pl.BlockSpec((R, d_in), lambda i: (i, 0)),
            pl.BlockSpec((d_in, hid), lambda i: (0, 0)),
            pl.BlockSpec((hid,), lambda i: (0,)),
            pl.BlockSpec((hid,), lambda i: (0,)),
        ],
        out_specs=[
            pl.BlockSpec((R, hid), lambda i: (i, 0)),
            pl.BlockSpec((R,), lambda i: (i,)),
            pl.BlockSpec((R,), lambda i: (i,)),
            pl.BlockSpec((128,), lambda i: (0,)),
        ],
        out_shape=[
            jax.ShapeDtypeStruct((N_PAD, hid), jnp.float32),
            jax.ShapeDtypeStruct((N_PAD,), jnp.float32),
            jax.ShapeDtypeStruct((N_PAD,), jnp.float32),
            jax.ShapeDtypeStruct((128,), jnp.float32),
        ],
        scratch_shapes=[pltpu.VMEM((2, 128), jnp.float32)],
    )(x_pad, W, a_src, a_dst)


def _sc_agg_body(nb, W,
                 h_hbm, es_hbm, ed_hbm, src_hbm, dst_hbm, cv_hbm,
                 out_hbm, wout_hbm,
                 srcb, dstb, dsts, rows, wbuf, esg, edg, cbuf,
                 out_acc, wacc, es_sh, ed_sh, sem_i, sem_g, sem_s):
    cid = lax.axis_index("c")
    sid = lax.axis_index("s")
    wid = cid * NS + sid
    stripe = N_PAD // NS
    soff = sid * stripe
    zi16 = jnp.zeros((L,), jnp.int32)
    iota16 = lax.iota(jnp.int32, L)

    pltpu.sync_copy(cv_hbm.at[pl.ds(0, L)], cbuf)
    pltpu.sync_copy(es_hbm.at[pl.ds(soff, stripe)],
                    es_sh.at[pl.ds(soff, stripe)])
    pltpu.sync_copy(ed_hbm.at[pl.ds(soff, stripe)],
                    ed_sh.at[pl.ds(soff, stripe)])

    zero = jnp.zeros((L,), jnp.float32)

    def _zrow(j, _):
        for k in range(W // L):
            rows[0, j, pl.ds(k * L, L)] = zero
        wbuf[0, j, pl.ds(0, L)] = zero
        wbuf[1, j, pl.ds(0, L)] = zero
        return 0

    lax.fori_loop(0, B, _zrow, 0)
    for k in range(stripe // ZC):
        pltpu.sync_copy(rows.at[0].at[pl.ds(0, ZC)],
                        out_acc.at[pl.ds(soff + k * ZC, ZC)])
        pltpu.sync_copy(wbuf.at[0].at[pl.ds(0, ZC)],
                        wacc.at[pl.ds(soff + k * ZC, ZC)])
    plsc.subcore_barrier()

    cv = cbuf[pl.ds(0, L)]

    def fire_idx(b, p):
        pltpu.async_copy(src_hbm.at[wid].at[b], srcb.at[p], sem_i)
        pltpu.async_copy(dst_hbm.at[wid].at[b], dstb.at[p], sem_i)

    def wait_idx(p):
        pltpu.make_async_copy(src_hbm.at[wid].at[0], srcb.at[p], sem_i).wait()
        pltpu.make_async_copy(dst_hbm.at[wid].at[0], dstb.at[p], sem_i).wait()

    def fire_gath(p):
        pltpu.async_copy(h_hbm.at[srcb.at[p]], rows.at[p], sem_g)
        pltpu.async_copy(es_sh.at[srcb.at[p]], esg.at[p], sem_g)
        pltpu.async_copy(ed_sh.at[dstb.at[p]], edg.at[p], sem_g)

    def wait_gath(p):
        pltpu.make_async_copy(h_hbm.at[pl.ds(0, B)], rows.at[p], sem_g).wait()
        pltpu.make_async_copy(es_hbm.at[pl.ds(0, B)], esg.at[p], sem_g).wait()
        pltpu.make_async_copy(ed_hbm.at[pl.ds(0, B)], edg.at[p], sem_g).wait()

    def fire_scat(p):
        pltpu.async_copy(rows.at[p], out_acc.at[dsts.at[p]], sem_s, add=True)
        pltpu.async_copy(wbuf.at[p], wacc.at[dsts.at[p]], sem_s, add=True)

    def wait_scat(p):
        pltpu.make_async_copy(rows.at[p], out_acc.at[pl.ds(0, B)],
                              sem_s).wait()
        pltpu.make_async_copy(wbuf.at[p], wacc.at[pl.ds(0, B)],
                              sem_s).wait()

    def _phase(b, p):
        pltpu.sync_copy(src_hbm.at[wid].at[b], srcb.at[p])
        pltpu.sync_copy(dst_hbm.at[wid].at[b], dstb.at[p])
        pltpu.async_copy(h_hbm.at[srcb.at[p]], rows.at[p], sem_g).wait()
        pltpu.async_copy(es_sh.at[srcb.at[p]], esg.at[p], sem_g).wait()
        pltpu.async_copy(ed_sh.at[dstb.at[p]], edg.at[p], sem_g).wait()
        for g in range(B // L):
            sv = esg.at[p][pl.ds(g * L, L)]
            dv = edg.at[p][pl.ds(g * L, L)]
            ee = jnp.exp(_leaky(sv + dv) - cv)
            plsc.store_scatter(wbuf.at[p], [g * L + iota16, zi16], ee)
        for g in range(B // L):
            dsts[p, pl.ds(g * L, L)] = dstb[p, pl.ds(g * L, L)]

        def _edge(jj, _):
            for u in range(8):
                j = jj * 8 + u
                wv = plsc.load_gather(
                    wbuf.at[p], [jnp.full((L,), j, jnp.int32), zi16])
                for k in range(W // L):
                    rows[p, j, pl.ds(k * L, L)] = (
                        rows[p, j, pl.ds(k * L, L)] * wv)
            return 0

        lax.fori_loop(0, B // 8, _edge, 0)

        pltpu.sync_copy(rows.at[p], out_acc.at[dsts.at[p]], add=True)
        pltpu.sync_copy(wbuf.at[p], wacc.at[dsts.at[p]], add=True)

    def _pair(i, _):
        _phase(2 * i, 0)
        _phase(2 * i + 1, 1)
        return 0

    lax.fori_loop(0, nb // 2, _pair, 0)
    plsc.subcore_barrier()

    for k in range(stripe // ZC):
        off = soff + k * ZC
        pltpu.sync_copy(out_acc.at[pl.ds(off, ZC)],
                        out_hbm.at[cid].at[pl.ds(off, ZC)])
        pltpu.sync_copy(wacc.at[pl.ds(off, ZC)],
                        wout_hbm.at[cid].at[pl.ds(off, ZC)])


def _sc_aggregate(h_pad, es, ed, src3, dst3, cv):
    W = h_pad.shape[1]
    nb = src3.shape[1]
    mesh = plsc.VectorSubcoreMesh(core_axis_name="c", subcore_axis_name="s")
    body = functools.partial(_sc_agg_body, nb, W)
    return pl.kernel(
        body,
        out_type=[
            jax.ShapeDtypeStruct((NC, N_PAD, W), jnp.float32),
            jax.ShapeDtypeStruct((NC, N_PAD, L), jnp.float32),
        ],
        mesh=mesh,
        compiler_params=pltpu.CompilerParams(needs_layout_passes=False,
                                             use_tc_tiling_on_sc=False),
        scratch_types=[
            pltpu.VMEM((2, B), jnp.int32),
            pltpu.VMEM((2, B), jnp.int32),
            pltpu.VMEM((2, B), jnp.int32),
            pltpu.VMEM((2, B, W), jnp.float32),
            pltpu.VMEM((2, B, L), jnp.float32),
            pltpu.VMEM((2, B), jnp.float32),
            pltpu.VMEM((2, B), jnp.float32),
            pltpu.VMEM((L,), jnp.float32),
            pltpu.VMEM_SHARED((N_PAD, W), jnp.float32),
            pltpu.VMEM_SHARED((N_PAD, L), jnp.float32),
            pltpu.VMEM_SHARED((N_PAD,), jnp.float32),
            pltpu.VMEM_SHARED((N_PAD,), jnp.float32),
            pltpu.SemaphoreType.DMA,
            pltpu.SemaphoreType.DMA,
            pltpu.SemaphoreType.DMA,
        ],
    )(h_pad, es, ed, src3, dst3, cv)


def _k2_body(grid, s_ref, d_ref, b1_ref, w2_ref, a2s_ref, a2d_ref,
             h2_ref, es_ref, ed_ref, cv_ref, acc_ref):
    i = pl.program_id(0)
    num = s_ref[0] + s_ref[1]
    den = (d_ref[0] + d_ref[1])[:, 0:1]
    den = jnp.where(den > 0, den, 1.0)
    h1 = jax.nn.relu(num / den + b1_ref[...][None, :])
    h2 = jnp.dot(h1, w2_ref[...], preferred_element_type=jnp.float32)
    h2_ref[...] = h2
    es = jnp.sum(h2 * a2s_ref[...][None, :], axis=1)
    ed = jnp.sum(h2 * a2d_ref[...][None, :], axis=1)
    es_ref[...] = es
    ed_ref[...] = ed
    _track_shift(i, grid, es, ed, cv_ref, acc_ref)


def _dense2(S1, D1, b1, W2p, a2s_p, a2d_p):
    hid = W2p.shape[0]
    lw = W2p.shape[1]
    R = 512
    grid = N_PAD // R
    return pl.pallas_call(
        functools.partial(_k2_body, grid),
        grid=(grid,),
        in_specs=[
            pl.BlockSpec((NC, R, hid), lambda i: (0, i, 0)),
            pl.BlockSpec((NC, R, L), lambda i: (0, i, 0)),
            pl.BlockSpec((hid,), lambda i: (0,)),
            pl.BlockSpec((hid, lw), lambda i: (0, 0)),
            pl.BlockSpec((lw,), lambda i: (0,)),
            pl.BlockSpec((lw,), lambda i: (0,)),
        ],
        out_specs=[
            pl.BlockSpec((R, lw), lambda i: (i, 0)),
            pl.BlockSpec((R,), lambda i: (i,)),
            pl.BlockSpec((R,), lambda i: (i,)),
            pl.BlockSpec((128,), lambda i: (0,)),
        ],
        out_shape=[
            jax.ShapeDtypeStruct((N_PAD, lw), jnp.float32),
            jax.ShapeDtypeStruct((N_PAD,), jnp.float32),
            jax.ShapeDtypeStruct((N_PAD,), jnp.float32),
            jax.ShapeDtypeStruct((128,), jnp.float32),
        ],
        scratch_shapes=[pltpu.VMEM((2, 128), jnp.float32)],
    )(S1, D1, b1, W2p, a2s_p, a2d_p)


def _k3_body(n_label, s_ref, d_ref, b2_ref, o_ref):
    num = (s_ref[0] + s_ref[1])[:, :n_label]
    den = (d_ref[0] + d_ref[1])[:, 0:1]
    den = jnp.where(den > 0, den, 1.0)
    logits = num / den + b2_ref[...][None, :]
    m = jnp.max(logits, axis=1, keepdims=True)
    z = logits - m
    o_ref[...] = z - jnp.log(jnp.sum(jnp.exp(z), axis=1, keepdims=True))


def _dense3(S2, D2, b2, n, n_label):
    lw = S2.shape[2]
    R = 512
    grid = N_PAD // R
    return pl.pallas_call(
        functools.partial(_k3_body, n_label),
        grid=(grid,),
        in_specs=[
            pl.BlockSpec((NC, R, lw), lambda i: (0, i, 0)),
            pl.BlockSpec((NC, R, L), lambda i: (0, i, 0)),
            pl.BlockSpec((n_label,), lambda i: (0,)),
        ],
        out_specs=pl.BlockSpec((R, n_label), lambda i: (i, 0)),
        out_shape=jax.ShapeDtypeStruct((n, n_label), jnp.float32),
    )(S2, D2, b2)


@jax.jit
def kernel(x, W1, a1_src, a1_dst, b1, W2, a2_src, a2_dst, b2, edge_index):
    n, d_in = x.shape
    hid = W1.shape[1]
    n_label = W2.shape[1]
    e = edge_index.shape[1]
    e_tot = e + n

    nb = -(-e_tot // (NW * B))
    nb += nb % 2
    e_pad = NW * nb * B
    loop = jnp.arange(n, dtype=jnp.int32)
    src = jnp.concatenate([edge_index[0].astype(jnp.int32), loop])
    dst = jnp.concatenate([edge_index[1].astype(jnp.int32), loop])
    pad_n = jnp.full((e_pad - e_tot,), n, jnp.int32)
    src3 = jnp.concatenate([src, pad_n]).reshape(NW, nb, B)
    dst3 = jnp.concatenate([dst, pad_n]).reshape(NW, nb, B)
    x_pad = jnp.zeros((N_PAD, d_in), jnp.float32).at[:n].set(x)

    h1, es1, ed1, cv1 = _dense1(x_pad, W1, a1_src, a1_dst)
    S1, D1 = _sc_aggregate(h1, es1, ed1, src3, dst3, cv1)

    lw = 48
    W2p = jnp.zeros((hid, lw), jnp.float32).at[:, :n_label].set(W2)
    a2s_p = jnp.zeros((lw,), jnp.float32).at[:n_label].set(a2_src)
    a2d_p = jnp.zeros((lw,), jnp.float32).at[:n_label].set(a2_dst)
    h2, es2, ed2, cv2 = _dense2(S1, D1, b1, W2p, a2s_p, a2d_p)

    S2, D2 = _sc_aggregate(h2, es2, ed2, src3, dst3, cv2)
    return _dense3(S2, D2, b2, n, n_label)


# speedup vs baseline: 23.4298x; 1.3068x over previous
import functools
import jax
import jax.numpy as jnp
from jax import lax
from jax.experimental import pallas as pl
from jax.experimental.pallas import tpu as pltpu
from jax.experimental.pallas import tpu_sc as plsc

NC = 2
NS = 16
NW = NC * NS
L = 16
B = 112
N_PAD = 10240
ZC = 80


def _leaky(t):
    return jnp.where(t >= 0, t, 0.2 * t)


def _track_shift(i, grid, es, ed, cv_ref, acc_ref):
    m_es = jnp.max(es)
    m_ed = jnp.max(ed)
    first = i == 0
    acc_ref[0, :] = jnp.where(first, jnp.full((128,), m_es),
                              jnp.maximum(acc_ref[0, :], m_es))
    acc_ref[1, :] = jnp.where(first, jnp.full((128,), m_ed),
                              jnp.maximum(acc_ref[1, :], m_ed))

    @pl.when(i == grid - 1)
    def _():
        cv_ref[...] = _leaky(acc_ref[0, :] + acc_ref[1, :])


def _k1_body(grid, x_ref, w_ref, asrc_ref, adst_ref,
             h_ref, es_ref, ed_ref, cv_ref, acc_ref):
    i = pl.program_id(0)
    h = jnp.dot(x_ref[...], w_ref[...], preferred_element_type=jnp.float32)
    h_ref[...] = h
    es = jnp.sum(h * asrc_ref[...][None, :], axis=1)
    ed = jnp.sum(h * adst_ref[...][None, :], axis=1)
    es_ref[...] = es
    ed_ref[...] = ed
    _track_shift(i, grid, es, ed, cv_ref, acc_ref)


def _dense1(x_pad, W, a_src, a_dst):
    d_in = x_pad.shape[1]
    hid = W.shape[1]
    R = 512
    grid = N_PAD // R
    return pl.pallas_call(
        functools.partial(_k1_body, grid),
        grid=(grid,),
        in_specs=[
            pl.BlockSpec((R, d_in), lambda i: (i, 0)),
            pl.BlockSpec((d_in, hid), lambda i: (0, 0)),
            pl.BlockSpec((hid,), lambda i: (0,)),
            pl.BlockSpec((hid,), lambda i: (0,)),
        ],
        out_specs=[
            pl.BlockSpec((R, hid), lambda i: (i, 0)),
            pl.BlockSpec((R,), lambda i: (i,)),
            pl.BlockSpec((R,), lambda i: (i,)),
            pl.BlockSpec((128,), lambda i: (0,)),
        ],
        out_shape=[
            jax.ShapeDtypeStruct((N_PAD, hid), jnp.float32),
            jax.ShapeDtypeStruct((N_PAD,), jnp.float32),
            jax.ShapeDtypeStruct((N_PAD,), jnp.float32),
            jax.ShapeDtypeStruct((128,), jnp.float32),
        ],
        scratch_shapes=[pltpu.VMEM((2, 128), jnp.float32)],
    )(x_pad, W, a_src, a_dst)


def _sc_agg_body(nb, W,
                 h_hbm, es_hbm, ed_hbm, src_hbm, dst_hbm, cv_hbm,
                 out_hbm, wout_hbm,
                 srcb, dstb, rows, wbuf, esg, edg, cbuf,
                 out_acc, wacc, es_sh, ed_sh, sem_g, sem_e, sem_s):
    cid = lax.axis_index("c")
    sid = lax.axis_index("s")
    wid = cid * NS + sid
    stripe = N_PAD // NS
    soff = sid * stripe
    zi16 = jnp.zeros((L,), jnp.int32)
    iota16 = lax.iota(jnp.int32, L)

    pltpu.sync_copy(cv_hbm.at[pl.ds(0, L)], cbuf)
    pltpu.sync_copy(es_hbm.at[pl.ds(soff, stripe)],
                    es_sh.at[pl.ds(soff, stripe)])
    pltpu.sync_copy(ed_hbm.at[pl.ds(soff, stripe)],
                    ed_sh.at[pl.ds(soff, stripe)])

    zero = jnp.zeros((L,), jnp.float32)

    def _zrow(j, _):
        for k in range(W // L):
            rows[0, j, pl.ds(k * L, L)] = zero
        wbuf[0, j, pl.ds(0, L)] = zero
        wbuf[1, j, pl.ds(0, L)] = zero
        return 0

    lax.fori_loop(0, B, _zrow, 0)
    for k in range(stripe // ZC):
        pltpu.sync_copy(rows.at[0].at[pl.ds(0, ZC)],
                        out_acc.at[pl.ds(soff + k * ZC, ZC)])
        pltpu.sync_copy(wbuf.at[0].at[pl.ds(0, ZC)],
                        wacc.at[pl.ds(soff + k * ZC, ZC)])
    plsc.subcore_barrier()

    cv = cbuf[pl.ds(0, L)]

    def _fire_gath(b, p):
        pltpu.sync_copy(src_hbm.at[wid].at[b], srcb.at[p])
        pltpu.sync_copy(dst_hbm.at[wid].at[b], dstb.at[p])
        dr = pltpu.async_copy(h_hbm.at[srcb.at[p]], rows.at[p], sem_g)
        de = pltpu.async_copy(es_sh.at[srcb.at[p]], esg.at[p], sem_e)
        dd = pltpu.async_copy(ed_sh.at[dstb.at[p]], edg.at[p], sem_e)
        return dr, de, dd

    def _compute(p):
        for g in range(B // L):
            sv = esg.at[p][pl.ds(g * L, L)]
            dv = edg.at[p][pl.ds(g * L, L)]
            ee = jnp.exp(_leaky(sv + dv) - cv)
            plsc.store_scatter(wbuf.at[p], [g * L + iota16, zi16], ee)

        def _edge(jj, _):
            for u in range(8):
                j = jj * 8 + u
                wv = plsc.load_gather(
                    wbuf.at[p], [jnp.full((L,), j, jnp.int32), zi16])
                for k in range(W // L):
                    rows[p, j, pl.ds(k * L, L)] = (
                        rows[p, j, pl.ds(k * L, L)] * wv)
            return 0

        lax.fori_loop(0, B // 8, _edge, 0)

    def _fire_scat(p):
        s1 = pltpu.async_copy(rows.at[p], out_acc.at[dstb.at[p]],
                              sem_s, add=True)
        s2 = pltpu.async_copy(wbuf.at[p], wacc.at[dstb.at[p]],
                              sem_s, add=True)
        return s1, s2

    def _pair(i, _):
        g0 = _fire_gath(2 * i, 0)
        g1 = _fire_gath(2 * i + 1, 1)
        for d in g0:
            d.wait()
        _compute(0)
        s0 = _fire_scat(0)
        for d in g1:
            d.wait()
        _compute(1)
        s1 = _fire_scat(1)
        for d in s0 + s1:
            d.wait()
        return 0

    lax.fori_loop(0, nb // 2, _pair, 0)
    plsc.subcore_barrier()

    for k in range(stripe // ZC):
        off = soff + k * ZC
        pltpu.sync_copy(out_acc.at[pl.ds(off, ZC)],
                        out_hbm.at[cid].at[pl.ds(off, ZC)])
        pltpu.sync_copy(wacc.at[pl.ds(off, ZC)],
                        wout_hbm.at[cid].at[pl.ds(off, ZC)])


def _sc_aggregate(h_pad, es, ed, src3, dst3, cv):
    W = h_pad.shape[1]
    nb = src3.shape[1]
    mesh = plsc.VectorSubcoreMesh(core_axis_name="c", subcore_axis_name="s")
    body = functools.partial(_sc_agg_body, nb, W)
    return pl.kernel(
        body,
        out_type=[
            jax.ShapeDtypeStruct((NC, N_PAD, W), jnp.float32),
            jax.ShapeDtypeStruct((NC, N_PAD, L), jnp.float32),
        ],
        mesh=mesh,
        compiler_params=pltpu.CompilerParams(needs_layout_passes=False,
                                             use_tc_tiling_on_sc=False),
        scratch_types=[
            pltpu.VMEM((2, B), jnp.int32),
            pltpu.VMEM((2, B), jnp.int32),
            pltpu.VMEM((2, B, W), jnp.float32),
            pltpu.VMEM((2, B, L), jnp.float32),
            pltpu.VMEM((2, B), jnp.float32),
            pltpu.VMEM((2, B), jnp.float32),
            pltpu.VMEM((L,), jnp.float32),
            pltpu.VMEM_SHARED((N_PAD, W), jnp.float32),
            pltpu.VMEM_SHARED((N_PAD, L), jnp.float32),
            pltpu.VMEM_SHARED((N_PAD,), jnp.float32),
            pltpu.VMEM_SHARED((N_PAD,), jnp.float32),
            pltpu.SemaphoreType.DMA,
            pltpu.SemaphoreType.DMA,
            pltpu.SemaphoreType.DMA,
        ],
    )(h_pad, es, ed, src3, dst3, cv)


def _k2_body(grid, s_ref, d_ref, b1_ref, w2_ref, a2s_ref, a2d_ref,
             h2_ref, es_ref, ed_ref, cv_ref, acc_ref):
    i = pl.program_id(0)
    num = s_ref[0] + s_ref[1]
    den = (d_ref[0] + d_ref[1])[:, 0:1]
    den = jnp.where(den > 0, den, 1.0)
    h1 = jax.nn.relu(num / den + b1_ref[...][None, :])
    h2 = jnp.dot(h1, w2_ref[...], preferred_element_type=jnp.float32)
    h2_ref[...] = h2
    es = jnp.sum(h2 * a2s_ref[...][None, :], axis=1)
    ed = jnp.sum(h2 * a2d_ref[...][None, :], axis=1)
    es_ref[...] = es
    ed_ref[...] = ed
    _track_shift(i, grid, es, ed, cv_ref, acc_ref)


def _dense2(S1, D1, b1, W2p, a2s_p, a2d_p):
    hid = W2p.shape[0]
    lw = W2p.shape[1]
    R = 512
    grid = N_PAD // R
    return pl.pallas_call(
        functools.partial(_k2_body, grid),
        grid=(grid,),
        in_specs=[
            pl.BlockSpec((NC, R, hid), lambda i: (0, i, 0)),
            pl.BlockSpec((NC, R, L), lambda i: (0, i, 0)),
            pl.BlockSpec((hid,), lambda i: (0,)),
            pl.BlockSpec((hid, lw), lambda i: (0, 0)),
            pl.BlockSpec((lw,), lambda i: (0,)),
            pl.BlockSpec((lw,), lambda i: (0,)),
        ],
        out_specs=[
            pl.BlockSpec((R, lw), lambda i: (i, 0)),
            pl.BlockSpec((R,), lambda i: (i,)),
            pl.BlockSpec((R,), lambda i: (i,)),
            pl.BlockSpec((128,), lambda i: (0,)),
        ],
        out_shape=[
            jax.ShapeDtypeStruct((N_PAD, lw), jnp.float32),
            jax.ShapeDtypeStruct((N_PAD,), jnp.float32),
            jax.ShapeDtypeStruct((N_PAD,), jnp.float32),
            jax.ShapeDtypeStruct((128,), jnp.float32),
        ],
        scratch_shapes=[pltpu.VMEM((2, 128), jnp.float32)],
    )(S1, D1, b1, W2p, a2s_p, a2d_p)


def _k3_body(n_label, s_ref, d_ref, b2_ref, o_ref):
    num = (s_ref[0] + s_ref[1])[:, :n_label]
    den = (d_ref[0] + d_ref[1])[:, 0:1]
    den = jnp.where(den > 0, den, 1.0)
    logits = num / den + b2_ref[...][None, :]
    m = jnp.max(logits, axis=1, keepdims=True)
    z = logits - m
    o_ref[...] = z - jnp.log(jnp.sum(jnp.exp(z), axis=1, keepdims=True))


def _dense3(S2, D2, b2, n, n_label):
    lw = S2.shape[2]
    R = 512
    grid = N_PAD // R
    return pl.pallas_call(
        functools.partial(_k3_body, n_label),
        grid=(grid,),
        in_specs=[
            pl.BlockSpec((NC, R, lw), lambda i: (0, i, 0)),
            pl.BlockSpec((NC, R, L), lambda i: (0, i, 0)),
            pl.BlockSpec((n_label,), lambda i: (0,)),
        ],
        out_specs=pl.BlockSpec((R, n_label), lambda i: (i, 0)),
        out_shape=jax.ShapeDtypeStruct((n, n_label), jnp.float32),
    )(S2, D2, b2)


@jax.jit
def kernel(x, W1, a1_src, a1_dst, b1, W2, a2_src, a2_dst, b2, edge_index):
    n, d_in = x.shape
    hid = W1.shape[1]
    n_label = W2.shape[1]
    e = edge_index.shape[1]
    e_tot = e + n

    nb = -(-e_tot // (NW * B))
    nb += nb % 2
    e_pad = NW * nb * B
    loop = jnp.arange(n, dtype=jnp.int32)
    src = jnp.concatenate([edge_index[0].astype(jnp.int32), loop])
    dst = jnp.concatenate([edge_index[1].astype(jnp.int32), loop])
    pad_n = jnp.full((e_pad - e_tot,), n, jnp.int32)
    src3 = jnp.concatenate([src, pad_n]).reshape(NW, nb, B)
    dst3 = jnp.concatenate([dst, pad_n]).reshape(NW, nb, B)
    x_pad = jnp.zeros((N_PAD, d_in), jnp.float32).at[:n].set(x)

    h1, es1, ed1, cv1 = _dense1(x_pad, W1, a1_src, a1_dst)
    S1, D1 = _sc_aggregate(h1, es1, ed1, src3, dst3, cv1)

    lw = 48
    W2p = jnp.zeros((hid, lw), jnp.float32).at[:, :n_label].set(W2)
    a2s_p = jnp.zeros((lw,), jnp.float32).at[:n_label].set(a2_src)
    a2d_p = jnp.zeros((lw,), jnp.float32).at[:n_label].set(a2_dst)
    h2, es2, ed2, cv2 = _dense2(S1, D1, b1, W2p, a2s_p, a2d_p)

    S2, D2 = _sc_aggregate(h2, es2, ed2, src3, dst3, cv2)
    return _dense3(S2, D2, b2, n, n_label)
